# Initial kernel scaffold; baseline (speedup 1.0000x reference)
import jax
import jax.numpy as jnp
from jax.experimental import pallas as pl


def kernel(x_embed, edge_index, batch, mask, alias_session, noise, W_gcn, b_gcn, W2, b2):
    raise NotImplementedError("write your pallas kernel here")


# speedup vs baseline: 4.0773x; 4.0773x over previous
import functools

import jax
import jax.numpy as jnp
from jax import lax
from jax.experimental import pallas as pl
from jax.experimental.pallas import tpu as pltpu
from jax.experimental.pallas import tpu_sc as plsc

F32 = jnp.float32
ALPHA = 0.2
TUA = 0.2
EPS = 0.1

NC = 2
NS = 16
NW = NC * NS


def _sc_mesh():
    return plsc.VectorSubcoreMesh(
        core_axis_name="c", subcore_axis_name="s", num_cores=NC, num_subcores=NS
    )


def _make_deg_kernel(nrows, nch):
    @functools.partial(
        pl.kernel,
        out_type=jax.ShapeDtypeStruct((NC, nrows, 128), F32),
        mesh=_sc_mesh(),
        scratch_types=[
            pltpu.VMEM((nch, 128), jnp.int32),
            pltpu.VMEM((128, 128), F32),
            pltpu.VMEM_SHARED((nrows, 128), F32),
            pltpu.SemaphoreType.DMA,
        ],
    )
    def deg_k(dst_hbm, ones_hbm, zeros_hbm, out_hbm, idx_v, ones_v, acc_sh, sem):
        c = lax.axis_index("c")
        s = lax.axis_index("s")
        w = c * NS + s
        rpt = nrows // NS
        pltpu.sync_copy(zeros_hbm.at[pl.ds(s * rpt, rpt)], acc_sh.at[pl.ds(s * rpt, rpt)])
        pltpu.sync_copy(ones_hbm, ones_v)
        pltpu.sync_copy(dst_hbm.at[pl.ds(w * nch, nch)], idx_v)
        plsc.subcore_barrier()

        def body(j, carry):
            pltpu.sync_copy(ones_v, acc_sh.at[idx_v.at[j]], add=True)
            return carry

        lax.fori_loop(0, nch, body, 0)
        plsc.subcore_barrier()
        pltpu.sync_copy(acc_sh.at[pl.ds(s * rpt, rpt)], out_hbm.at[c, pl.ds(s * rpt, rpt)])

    return deg_k


def _make_conv_kernel(nrows, nch, hchunks):
    @functools.partial(
        pl.kernel,
        out_type=jax.ShapeDtypeStruct((NC, nrows, 128 * hchunks), F32),
        mesh=_sc_mesh(),
        scratch_types=[
            pltpu.VMEM((nch, 128), jnp.int32),
            pltpu.VMEM((nch, 128), jnp.int32),
            pltpu.VMEM((128, 128), F32),
            pltpu.VMEM_SHARED((nrows, 128), F32),
            pltpu.SemaphoreType.DMA,
        ],
    )
    def conv_k(src_hbm, dst_hbm, hs0, hs1, hs2, hs3, zeros_hbm, out_hbm,
               idx_s, idx_d, rows_v, acc_sh, sem):
        c = lax.axis_index("c")
        s = lax.axis_index("s")
        w = c * NS + s
        rpt = nrows // NS
        pltpu.sync_copy(src_hbm.at[pl.ds(w * nch, nch)], idx_s)
        pltpu.sync_copy(dst_hbm.at[pl.ds(w * nch, nch)], idx_d)
        hs_all = (hs0, hs1, hs2, hs3)
        for k in range(hchunks):
            hs_k = hs_all[k]
            pltpu.sync_copy(zeros_hbm.at[pl.ds(s * rpt, rpt)],
                            acc_sh.at[pl.ds(s * rpt, rpt)])
            plsc.subcore_barrier()

            def body(j, carry):
                pltpu.async_copy(hs_k.at[idx_s.at[j]], rows_v, sem).wait()
                pltpu.sync_copy(rows_v, acc_sh.at[idx_d.at[j]], add=True)
                return carry

            lax.fori_loop(0, nch, body, 0)
            plsc.subcore_barrier()
            pltpu.sync_copy(
                acc_sh.at[pl.ds(s * rpt, rpt)],
                out_hbm.at[c, pl.ds(s * rpt, rpt), pl.ds(k * 128, 128)],
            )
            plsc.subcore_barrier()

    return conv_k


def _make_gather_kernel(n_src_rows, d, gpt, ngpad):
    @functools.partial(
        pl.kernel,
        out_type=jax.ShapeDtypeStruct((ngpad, d), F32),
        mesh=_sc_mesh(),
        scratch_types=[
            pltpu.VMEM((8, 128), jnp.int32),
            pltpu.VMEM((128, d), F32),
            pltpu.SemaphoreType.DMA,
        ],
        name="session_gather",
    )
    def gather_k(ne_hbm, gidx_hbm, out_hbm, idx_v, rows_v, sem):
        c = lax.axis_index("c")
        s = lax.axis_index("s")
        w = c * NS + s
        pltpu.sync_copy(gidx_hbm.at[pl.ds(w * 8, 8)], idx_v)
        for j in range(gpt):
            pltpu.async_copy(ne_hbm.at[idx_v.at[j]], rows_v, sem).wait()
            pltpu.sync_copy(rows_v, out_hbm.at[pl.ds((w * gpt + j) * 128, 128)])

    return gather_k


def _prep_body(x_ref, nz_ref, w_ref, deg_ref,
               hs0, hs1, hs2, hs3, xcl_ref, dis_ref):
    x = x_ref[...]
    nz = nz_ref[...]
    nn = jnp.sqrt(jnp.sum(nz * nz, axis=1, keepdims=True))
    xcl = x + nz / jnp.maximum(nn, 1e-12) * EPS
    degs = deg_ref[0] + deg_ref[1]
    dis = lax.rsqrt(degs[:, 0:1] + 1.0)
    w = w_ref[...]
    h1 = jnp.dot(x, w, preferred_element_type=F32) * dis
    h2 = jnp.dot(xcl, w, preferred_element_type=F32) * dis
    hs0[...] = h1[:, :128]
    hs1[...] = h1[:, 128:]
    hs2[...] = h2[:, :128]
    hs3[...] = h2[:, 128:]
    xcl_ref[...] = xcl
    dis_ref[...] = jnp.broadcast_to(dis, dis_ref.shape)


def _combine_body(acc_ref, hs0, hs1, hs2, hs3, x_ref, xcl_ref, dis_ref, b_ref,
                  ne_ref, v1_ref, v2_ref, pos_ref):
    d = x_ref.shape[1]
    a = acc_ref[0] + acc_ref[1]
    dis = dis_ref[:, 0:1]
    b = b_ref[...]
    h1 = jnp.concatenate([hs0[...], hs1[...]], axis=1)
    h2 = jnp.concatenate([hs2[...], hs3[...]], axis=1)
    conv1 = (a[:, :d] + h1) * dis + b
    conv2 = (a[:, d:] + h2) * dis + b
    ne = jnp.where(conv1 >= 0, conv1, ALPHA * conv1) + x_ref[...]
    necl = jnp.where(conv2 >= 0, conv2, ALPHA * conv2) + xcl_ref[...]
    n1 = jnp.sqrt(jnp.sum(ne * ne, axis=1, keepdims=True))
    v1 = ne / jnp.maximum(n1, 1e-12)
    n2 = jnp.sqrt(jnp.sum(necl * necl, axis=1, keepdims=True))
    v2 = necl / jnp.maximum(n2, 1e-12)
    ne_ref[...] = ne
    v1_ref[...] = v1
    v2_ref[...] = v2
    pos_ref[...] = jnp.broadcast_to(jnp.sum(v1 * v2, axis=1, keepdims=True),
                                    pos_ref.shape)


def _make_nce_body(n_total):
    def _nce_body(v1_ref, v2_ref, pos_ref, out_ref, srow, accs):
        i = pl.program_id(0)
        j = pl.program_id(1)
        ni = pl.num_programs(0)
        nj = pl.num_programs(1)
        sim = lax.dot_general(v1_ref[...], v2_ref[...],
                              (((1,), (1,)), ((), ())),
                              preferred_element_type=F32)
        e = jnp.exp(sim * (1.0 / TUA))
        rs = jnp.sum(e, axis=1, keepdims=True)

        @pl.when(j == 0)
        def _():
            srow[...] = jnp.broadcast_to(rs, srow.shape)

        @pl.when(j > 0)
        def _():
            srow[...] += jnp.broadcast_to(rs, srow.shape)

        @pl.when(j == nj - 1)
        def _():
            part = jnp.sum(jnp.log(srow[:, 0:1]) - pos_ref[:, 0:1] * (1.0 / TUA))

            @pl.when(i == 0)
            def _():
                accs[0, 0] = part

            @pl.when(i > 0)
            def _():
                accs[0, 0] += part

            @pl.when(i == ni - 1)
            def _():
                out_ref[...] = jnp.broadcast_to(accs[0, 0] * (1.0 / n_total), (1, 1))

    return _nce_body


def _make_sess_body(n_sess_blk, sess_len):
    def _sess_body(seq_ref, mask_ref, w2_ref, b2_ref, out_ref):
        d = seq_ref.shape[1]
        sq = seq_ref[...] * mask_ref[:, 0:1]
        m3 = sq.reshape(n_sess_blk, sess_len, d)
        mean = jnp.sum(m3, axis=1) * (1.0 / sess_len)
        sm = lax.dot_general(mean, w2_ref[...], (((1,), (1,)), ((), ())),
                             preferred_element_type=F32) + b2_ref[...]
        sm = jnp.maximum(sm, 0.0)
        smb = jnp.broadcast_to(sm[:, None, :], (n_sess_blk, sess_len, d))
        out_ref[...] = sq + smb.reshape(n_sess_blk * sess_len, d)

    return _sess_body


def kernel(x_embed, edge_index, batch, mask, alias_session, noise,
           W_gcn, b_gcn, W2, b2):
    n, d = x_embed.shape
    e = edge_index.shape[1]
    bs, ls = alias_session.shape

    nrows = ((n + 1 + 16 * 8 * 5 - 1) // (16 * 8 * 5)) * (16 * 8 * 5)
    epad = ((e + NW * 128 - 1) // (NW * 128)) * (NW * 128)
    nch = epad // (NW * 128)
    hchunks = (2 * d) // 128

    src = edge_index[0]
    dst = edge_index[1]
    pad_e = epad - e
    srcp = jnp.concatenate([src, jnp.zeros((pad_e,), jnp.int32)]).reshape(epad // 128, 128)
    dstp = jnp.concatenate([dst, jnp.full((pad_e,), n, jnp.int32)]).reshape(epad // 128, 128)
    z128 = jnp.zeros((nrows, 128), F32)
    ones128 = jnp.ones((128, 128), F32)

    deg2 = _make_deg_kernel(nrows, nch)(dstp, ones128, z128)

    rb = 1000
    gridn = n // rb
    hs_sh = jax.ShapeDtypeStruct((n, 128), F32)
    hs0, hs1, hs2, hs3, xcl, disb = pl.pallas_call(
        _prep_body,
        grid=(gridn,),
        in_specs=[
            pl.BlockSpec((rb, d), lambda i: (i, 0)),
            pl.BlockSpec((rb, d), lambda i: (i, 0)),
            pl.BlockSpec((d, d), lambda i: (0, 0)),
            pl.BlockSpec((NC, rb, 128), lambda i: (0, i, 0)),
        ],
        out_specs=[
            pl.BlockSpec((rb, 128), lambda i: (i, 0)),
            pl.BlockSpec((rb, 128), lambda i: (i, 0)),
            pl.BlockSpec((rb, 128), lambda i: (i, 0)),
            pl.BlockSpec((rb, 128), lambda i: (i, 0)),
            pl.BlockSpec((rb, d), lambda i: (i, 0)),
            pl.BlockSpec((rb, 128), lambda i: (i, 0)),
        ],
        out_shape=[hs_sh, hs_sh, hs_sh, hs_sh,
                   jax.ShapeDtypeStruct((n, d), F32),
                   jax.ShapeDtypeStruct((n, 128), F32)],
    )(x_embed, noise, W_gcn, deg2)

    acc = _make_conv_kernel(nrows, nch, hchunks)(srcp, dstp, hs0, hs1, hs2, hs3, z128)

    ne, v1, v2, posb = pl.pallas_call(
        _combine_body,
        grid=(gridn,),
        in_specs=[
            pl.BlockSpec((NC, rb, 2 * d), lambda i: (0, i, 0)),
            pl.BlockSpec((rb, 128), lambda i: (i, 0)),
            pl.BlockSpec((rb, 128), lambda i: (i, 0)),
            pl.BlockSpec((rb, 128), lambda i: (i, 0)),
            pl.BlockSpec((rb, 128), lambda i: (i, 0)),
            pl.BlockSpec((rb, d), lambda i: (i, 0)),
            pl.BlockSpec((rb, d), lambda i: (i, 0)),
            pl.BlockSpec((rb, 128), lambda i: (i, 0)),
            pl.BlockSpec((1, d), lambda i: (0, 0)),
        ],
        out_specs=[
            pl.BlockSpec((rb, d), lambda i: (i, 0)),
            pl.BlockSpec((rb, d), lambda i: (i, 0)),
            pl.BlockSpec((rb, d), lambda i: (i, 0)),
            pl.BlockSpec((rb, 128), lambda i: (i, 0)),
        ],
        out_shape=[jax.ShapeDtypeStruct((n, d), F32),
                   jax.ShapeDtypeStruct((n, d), F32),
                   jax.ShapeDtypeStruct((n, d), F32),
                   jax.ShapeDtypeStruct((n, 128), F32)],
    )(acc, hs0, hs1, hs2, hs3, x_embed, xcl, disb, b_gcn.reshape(1, d))

    offsets = jnp.searchsorted(batch, jnp.arange(bs, dtype=batch.dtype)).astype(jnp.int32)
    gidx = (alias_session.astype(jnp.int32) + offsets[:, None]).reshape(-1)
    ng = bs * ls
    ngpad = ((ng + NW * 128 - 1) // (NW * 128)) * (NW * 128)
    gpt = ngpad // (NW * 128)
    g3 = jnp.concatenate([gidx, jnp.zeros((ngpad - ng,), jnp.int32)]).reshape(NW, gpt, 128)
    gidxp = jnp.zeros((NW, 8, 128), jnp.int32).at[:, :gpt].set(g3).reshape(NW * 8, 128)
    seq = _make_gather_kernel(n, d, gpt, ngpad)(ne, gidxp)

    loss = pl.pallas_call(
        _make_nce_body(n),
        grid=(gridn, gridn),
        in_specs=[
            pl.BlockSpec((rb, d), lambda i, j: (i, 0)),
            pl.BlockSpec((rb, d), lambda i, j: (j, 0)),
            pl.BlockSpec((rb, 128), lambda i, j: (i, 0)),
        ],
        out_specs=pl.BlockSpec((1, 1), lambda i, j: (0, 0)),
        out_shape=jax.ShapeDtypeStruct((1, 1), F32),
        scratch_shapes=[pltpu.VMEM((rb, 128), F32), pltpu.SMEM((1, 1), F32)],
    )(v1, v2, posb)

    sb = 20
    maskb = jnp.broadcast_to(mask.reshape(ng, 1), (ng, 128))
    spre = pl.pallas_call(
        _make_sess_body(sb, ls),
        grid=(bs // sb,),
        in_specs=[
            pl.BlockSpec((sb * ls, d), lambda i: (i, 0)),
            pl.BlockSpec((sb * ls, 128), lambda i: (i, 0)),
            pl.BlockSpec((d, d), lambda i: (0, 0)),
            pl.BlockSpec((1, d), lambda i: (0, 0)),
        ],
        out_specs=pl.BlockSpec((sb * ls, d), lambda i: (i, 0)),
        out_shape=jax.ShapeDtypeStruct((ng, d), F32),
    )(seq, maskb, W2, b2.reshape(1, d))

    return (spre.reshape(bs, ls, d), loss[0, 0])


# speedup vs baseline: 4.4440x; 1.0899x over previous
import functools

import jax
import jax.numpy as jnp
from jax import lax
from jax.experimental import pallas as pl
from jax.experimental.pallas import tpu as pltpu
from jax.experimental.pallas import tpu_sc as plsc

F32 = jnp.float32
ALPHA = 0.2
TUA = 0.2
EPS = 0.1

NC = 2
NS = 16
NW = NC * NS


def _sc_mesh():
    return plsc.VectorSubcoreMesh(
        core_axis_name="c", subcore_axis_name="s", num_cores=NC, num_subcores=NS
    )


def _make_deg_kernel(nrows, nch):
    @functools.partial(
        pl.kernel,
        out_type=jax.ShapeDtypeStruct((NC, nrows, 128), F32),
        mesh=_sc_mesh(),
        scratch_types=[
            pltpu.VMEM((nch, 128), jnp.int32),
            pltpu.VMEM((128, 128), F32),
            pltpu.VMEM_SHARED((nrows, 128), F32),
            pltpu.SemaphoreType.DMA,
        ],
    )
    def deg_k(dst_hbm, ones_hbm, zeros_hbm, out_hbm, idx_v, ones_v, acc_sh, sem):
        c = lax.axis_index("c")
        s = lax.axis_index("s")
        w = c * NS + s
        rpt = nrows // NS
        pltpu.sync_copy(zeros_hbm.at[pl.ds(s * rpt, rpt)], acc_sh.at[pl.ds(s * rpt, rpt)])
        pltpu.sync_copy(ones_hbm, ones_v)
        pltpu.sync_copy(dst_hbm.at[pl.ds(w * nch, nch)], idx_v)
        plsc.subcore_barrier()

        def body(j, carry):
            pltpu.sync_copy(ones_v, acc_sh.at[idx_v.at[j]], add=True)
            return carry

        lax.fori_loop(0, nch, body, 0)
        plsc.subcore_barrier()
        pltpu.sync_copy(acc_sh.at[pl.ds(s * rpt, rpt)], out_hbm.at[c, pl.ds(s * rpt, rpt)])

    return deg_k


def _make_conv_kernel(nrows, nch, hchunks):
    @functools.partial(
        pl.kernel,
        out_type=jax.ShapeDtypeStruct((NC, nrows, 128 * hchunks), F32),
        mesh=_sc_mesh(),
        scratch_types=[
            pltpu.VMEM((nch, 128), jnp.int32),
            pltpu.VMEM((nch, 128), jnp.int32),
            pltpu.VMEM((128, 128), F32),
            pltpu.VMEM((128, 128), F32),
            pltpu.VMEM_SHARED((nrows, 128), F32),
            pltpu.SemaphoreType.DMA,
            pltpu.SemaphoreType.DMA,
        ],
    )
    def conv_k(src_hbm, dst_hbm, hs0, hs1, hs2, hs3, zeros_hbm, out_hbm,
               idx_s, idx_d, buf0, buf1, acc_sh, sem0, sem1):
        c = lax.axis_index("c")
        s = lax.axis_index("s")
        w = c * NS + s
        rpt = nrows // NS
        pltpu.sync_copy(src_hbm.at[pl.ds(w * nch, nch)], idx_s)
        pltpu.sync_copy(dst_hbm.at[pl.ds(w * nch, nch)], idx_d)
        hs_all = (hs0, hs1, hs2, hs3)
        last = nch - 1
        for k in range(hchunks):
            hs_k = hs_all[k]
            pltpu.sync_copy(zeros_hbm.at[pl.ds(s * rpt, rpt)],
                            acc_sh.at[pl.ds(s * rpt, rpt)])
            plsc.subcore_barrier()
            pltpu.async_copy(hs_k.at[idx_s.at[0]], buf0, sem0)

            def body(jj, carry):
                j0 = 2 * jj
                j1 = j0 + 1
                pltpu.async_copy(hs_k.at[idx_s.at[j1]], buf1, sem1)
                pltpu.make_async_copy(hs_k.at[idx_s.at[j0]], buf0, sem0).wait()
                pltpu.sync_copy(buf0, acc_sh.at[idx_d.at[j0]], add=True)
                jn = jnp.minimum(j0 + 2, last)
                pltpu.async_copy(hs_k.at[idx_s.at[jn]], buf0, sem0)
                pltpu.make_async_copy(hs_k.at[idx_s.at[j1]], buf1, sem1).wait()
                pltpu.sync_copy(buf1, acc_sh.at[idx_d.at[j1]], add=True)
                return carry

            lax.fori_loop(0, nch // 2, body, 0)
            pltpu.make_async_copy(hs_k.at[idx_s.at[last]], buf0, sem0).wait()
            plsc.subcore_barrier()
            pltpu.sync_copy(
                acc_sh.at[pl.ds(s * rpt, rpt)],
                out_hbm.at[c, pl.ds(s * rpt, rpt), pl.ds(k * 128, 128)],
            )
            plsc.subcore_barrier()

    return conv_k


def _make_gather_kernel(n_src_rows, d, gpt, ngpad):
    @functools.partial(
        pl.kernel,
        out_type=jax.ShapeDtypeStruct((ngpad, d), F32),
        mesh=_sc_mesh(),
        scratch_types=[
            pltpu.VMEM((8, 128), jnp.int32),
            pltpu.VMEM((128, d), F32),
            pltpu.SemaphoreType.DMA,
        ],
        name="session_gather",
    )
    def gather_k(ne_hbm, gidx_hbm, out_hbm, idx_v, rows_v, sem):
        c = lax.axis_index("c")
        s = lax.axis_index("s")
        w = c * NS + s
        pltpu.sync_copy(gidx_hbm.at[pl.ds(w * 8, 8)], idx_v)
        for j in range(gpt):
            pltpu.async_copy(ne_hbm.at[idx_v.at[j]], rows_v, sem).wait()
            pltpu.sync_copy(rows_v, out_hbm.at[pl.ds((w * gpt + j) * 128, 128)])

    return gather_k


def _prep_body(x_ref, nz_ref, w_ref, deg_ref,
               hs0, hs1, hs2, hs3, xcl_ref, dis_ref):
    x = x_ref[...]
    nz = nz_ref[...]
    nn = jnp.sqrt(jnp.sum(nz * nz, axis=1, keepdims=True))
    xcl = x + nz / jnp.maximum(nn, 1e-12) * EPS
    degs = deg_ref[0] + deg_ref[1]
    dis = lax.rsqrt(degs[:, 0:1] + 1.0)
    w = w_ref[...]
    h1 = jnp.dot(x, w, preferred_element_type=F32) * dis
    h2 = jnp.dot(xcl, w, preferred_element_type=F32) * dis
    hs0[...] = h1[:, :128]
    hs1[...] = h1[:, 128:]
    hs2[...] = h2[:, :128]
    hs3[...] = h2[:, 128:]
    xcl_ref[...] = xcl
    dis_ref[...] = jnp.broadcast_to(dis, dis_ref.shape)


def _combine_body(acc_ref, hs0, hs1, hs2, hs3, x_ref, xcl_ref, dis_ref, b_ref,
                  ne_ref, v1_ref, v2_ref, pos_ref):
    d = x_ref.shape[1]
    a = acc_ref[0] + acc_ref[1]
    dis = dis_ref[:, 0:1]
    b = b_ref[...]
    h1 = jnp.concatenate([hs0[...], hs1[...]], axis=1)
    h2 = jnp.concatenate([hs2[...], hs3[...]], axis=1)
    conv1 = (a[:, :d] + h1) * dis + b
    conv2 = (a[:, d:] + h2) * dis + b
    ne = jnp.where(conv1 >= 0, conv1, ALPHA * conv1) + x_ref[...]
    necl = jnp.where(conv2 >= 0, conv2, ALPHA * conv2) + xcl_ref[...]
    n1 = jnp.sqrt(jnp.sum(ne * ne, axis=1, keepdims=True))
    v1 = ne / jnp.maximum(n1, 1e-12)
    n2 = jnp.sqrt(jnp.sum(necl * necl, axis=1, keepdims=True))
    v2 = necl / jnp.maximum(n2, 1e-12)
    ne_ref[...] = ne
    v1_ref[...] = v1
    v2_ref[...] = v2
    pos_ref[...] = jnp.broadcast_to(jnp.sum(v1 * v2, axis=1, keepdims=True),
                                    pos_ref.shape)


def _make_nce_body(n_total):
    def _nce_body(v1_ref, v2_ref, pos_ref, out_ref, srow, accs):
        i = pl.program_id(0)
        j = pl.program_id(1)
        ni = pl.num_programs(0)
        nj = pl.num_programs(1)
        sim = lax.dot_general(v1_ref[...], v2_ref[...],
                              (((1,), (1,)), ((), ())),
                              preferred_element_type=F32)
        e = jnp.exp(sim * (1.0 / TUA))
        rs = jnp.sum(e, axis=1, keepdims=True)

        @pl.when(j == 0)
        def _():
            srow[...] = jnp.broadcast_to(rs, srow.shape)

        @pl.when(j > 0)
        def _():
            srow[...] += jnp.broadcast_to(rs, srow.shape)

        @pl.when(j == nj - 1)
        def _():
            part = jnp.sum(jnp.log(srow[:, 0:1]) - pos_ref[:, 0:1] * (1.0 / TUA))

            @pl.when(i == 0)
            def _():
                accs[0, 0] = part

            @pl.when(i > 0)
            def _():
                accs[0, 0] += part

            @pl.when(i == ni - 1)
            def _():
                out_ref[...] = jnp.broadcast_to(accs[0, 0] * (1.0 / n_total), (1, 1))

    return _nce_body


def _make_sess_body(n_sess_blk, sess_len):
    def _sess_body(seq_ref, mask_ref, w2_ref, b2_ref, out_ref):
        d = seq_ref.shape[1]
        sq = seq_ref[...] * mask_ref[:, 0:1]
        m3 = sq.reshape(n_sess_blk, sess_len, d)
        mean = jnp.sum(m3, axis=1) * (1.0 / sess_len)
        sm = lax.dot_general(mean, w2_ref[...], (((1,), (1,)), ((), ())),
                             preferred_element_type=F32) + b2_ref[...]
        sm = jnp.maximum(sm, 0.0)
        smb = jnp.broadcast_to(sm[:, None, :], (n_sess_blk, sess_len, d))
        out_ref[...] = sq + smb.reshape(n_sess_blk * sess_len, d)

    return _sess_body


def kernel(x_embed, edge_index, batch, mask, alias_session, noise,
           W_gcn, b_gcn, W2, b2):
    n, d = x_embed.shape
    e = edge_index.shape[1]
    bs, ls = alias_session.shape

    nrows = ((n + 1 + 16 * 8 * 5 - 1) // (16 * 8 * 5)) * (16 * 8 * 5)
    epad = ((e + NW * 128 - 1) // (NW * 128)) * (NW * 128)
    nch = epad // (NW * 128)
    hchunks = (2 * d) // 128

    src = edge_index[0]
    dst = edge_index[1]
    pad_e = epad - e
    srcp = jnp.concatenate([src, jnp.zeros((pad_e,), jnp.int32)]).reshape(epad // 128, 128)
    dstp = jnp.concatenate([dst, jnp.full((pad_e,), n, jnp.int32)]).reshape(epad // 128, 128)
    z128 = jnp.zeros((nrows, 128), F32)
    ones128 = jnp.ones((128, 128), F32)

    deg2 = _make_deg_kernel(nrows, nch)(dstp, ones128, z128)

    rb = 1000
    gridn = n // rb
    hs_sh = jax.ShapeDtypeStruct((n, 128), F32)
    hs0, hs1, hs2, hs3, xcl, disb = pl.pallas_call(
        _prep_body,
        grid=(gridn,),
        in_specs=[
            pl.BlockSpec((rb, d), lambda i: (i, 0)),
            pl.BlockSpec((rb, d), lambda i: (i, 0)),
            pl.BlockSpec((d, d), lambda i: (0, 0)),
            pl.BlockSpec((NC, rb, 128), lambda i: (0, i, 0)),
        ],
        out_specs=[
            pl.BlockSpec((rb, 128), lambda i: (i, 0)),
            pl.BlockSpec((rb, 128), lambda i: (i, 0)),
            pl.BlockSpec((rb, 128), lambda i: (i, 0)),
            pl.BlockSpec((rb, 128), lambda i: (i, 0)),
            pl.BlockSpec((rb, d), lambda i: (i, 0)),
            pl.BlockSpec((rb, 128), lambda i: (i, 0)),
        ],
        out_shape=[hs_sh, hs_sh, hs_sh, hs_sh,
                   jax.ShapeDtypeStruct((n, d), F32),
                   jax.ShapeDtypeStruct((n, 128), F32)],
    )(x_embed, noise, W_gcn, deg2)

    acc = _make_conv_kernel(nrows, nch, hchunks)(srcp, dstp, hs0, hs1, hs2, hs3, z128)

    ne, v1, v2, posb = pl.pallas_call(
        _combine_body,
        grid=(gridn,),
        in_specs=[
            pl.BlockSpec((NC, rb, 2 * d), lambda i: (0, i, 0)),
            pl.BlockSpec((rb, 128), lambda i: (i, 0)),
            pl.BlockSpec((rb, 128), lambda i: (i, 0)),
            pl.BlockSpec((rb, 128), lambda i: (i, 0)),
            pl.BlockSpec((rb, 128), lambda i: (i, 0)),
            pl.BlockSpec((rb, d), lambda i: (i, 0)),
            pl.BlockSpec((rb, d), lambda i: (i, 0)),
            pl.BlockSpec((rb, 128), lambda i: (i, 0)),
            pl.BlockSpec((1, d), lambda i: (0, 0)),
        ],
        out_specs=[
            pl.BlockSpec((rb, d), lambda i: (i, 0)),
            pl.BlockSpec((rb, d), lambda i: (i, 0)),
            pl.BlockSpec((rb, d), lambda i: (i, 0)),
            pl.BlockSpec((rb, 128), lambda i: (i, 0)),
        ],
        out_shape=[jax.ShapeDtypeStruct((n, d), F32),
                   jax.ShapeDtypeStruct((n, d), F32),
                   jax.ShapeDtypeStruct((n, d), F32),
                   jax.ShapeDtypeStruct((n, 128), F32)],
    )(acc, hs0, hs1, hs2, hs3, x_embed, xcl, disb, b_gcn.reshape(1, d))

    offsets = jnp.searchsorted(batch, jnp.arange(bs, dtype=batch.dtype)).astype(jnp.int32)
    gidx = (alias_session.astype(jnp.int32) + offsets[:, None]).reshape(-1)
    ng = bs * ls
    ngpad = ((ng + NW * 128 - 1) // (NW * 128)) * (NW * 128)
    gpt = ngpad // (NW * 128)
    g3 = jnp.concatenate([gidx, jnp.zeros((ngpad - ng,), jnp.int32)]).reshape(NW, gpt, 128)
    gidxp = jnp.zeros((NW, 8, 128), jnp.int32).at[:, :gpt].set(g3).reshape(NW * 8, 128)
    seq = _make_gather_kernel(n, d, gpt, ngpad)(ne, gidxp)

    loss = pl.pallas_call(
        _make_nce_body(n),
        grid=(gridn, gridn),
        in_specs=[
            pl.BlockSpec((rb, d), lambda i, j: (i, 0)),
            pl.BlockSpec((rb, d), lambda i, j: (j, 0)),
            pl.BlockSpec((rb, 128), lambda i, j: (i, 0)),
        ],
        out_specs=pl.BlockSpec((1, 1), lambda i, j: (0, 0)),
        out_shape=jax.ShapeDtypeStruct((1, 1), F32),
        scratch_shapes=[pltpu.VMEM((rb, 128), F32), pltpu.SMEM((1, 1), F32)],
    )(v1, v2, posb)

    sb = 20
    maskb = jnp.broadcast_to(mask.reshape(ng, 1), (ng, 128))
    spre = pl.pallas_call(
        _make_sess_body(sb, ls),
        grid=(bs // sb,),
        in_specs=[
            pl.BlockSpec((sb * ls, d), lambda i: (i, 0)),
            pl.BlockSpec((sb * ls, 128), lambda i: (i, 0)),
            pl.BlockSpec((d, d), lambda i: (0, 0)),
            pl.BlockSpec((1, d), lambda i: (0, 0)),
        ],
        out_specs=pl.BlockSpec((sb * ls, d), lambda i: (i, 0)),
        out_shape=jax.ShapeDtypeStruct((ng, d), F32),
    )(seq, maskb, W2, b2.reshape(1, d))

    return (spre.reshape(bs, ls, d), loss[0, 0])
